# Initial kernel scaffold; baseline (speedup 1.0000x reference)
import jax
import jax.numpy as jnp
from jax.experimental import pallas as pl


def kernel(new_xyz, xyz):
    raise NotImplementedError("write your pallas kernel here")


# speedup vs baseline: 28.7304x; 28.7304x over previous
import dataclasses
import functools

import jax
import jax.numpy as jnp
from jax import lax
from jax.experimental import pallas as pl
from jax.experimental.pallas import tpu as pltpu
from jax.experimental.pallas import tpu_sc as plsc

RADIUS2 = 0.1 * 0.1
NSAMPLE = 64
B, S, N = 4, 1024, 16384
W = 16
NWORDS = N // W
ROWS = B * S
S_TILE = 256
NC = 2048
NWC = NC // W

NUM_WORKERS = 32
ROWS_PER = ROWS // NUM_WORKERS
BUF = 80


def _pack_kernel(q_ref, p_ref, a_ref, out_ref):
    q = q_ref[0]
    p = p_ref[0]
    qx, qy, qz = q[:, 0:1], q[:, 1:2], q[:, 2:3]
    px, py, pz = p[0:1, :], p[1:2, :], p[2:3, :]
    q2 = qx * qx + qy * qy + qz * qz
    p2 = px * px + py * py + pz * pz
    qb = q[:, 0:8].astype(jnp.bfloat16)
    pb = p.astype(jnp.bfloat16)
    qp = jnp.dot(qb, pb, preferred_element_type=jnp.float32)
    d2 = q2 + p2 - 2.0 * qp
    m = (d2 < RADIUS2).astype(jnp.bfloat16)
    w = jnp.dot(m, a_ref[...], preferred_element_type=jnp.float32)
    out_ref[0] = w.astype(jnp.int32)


def _pack(new_xyz, xyz):
    q = jnp.pad(new_xyz, ((0, 0), (0, 0), (0, 128 - 3)))
    p = jnp.pad(xyz.transpose(0, 2, 1), ((0, 0), (0, 8 - 3), (0, 0)))
    ii = jnp.arange(NC, dtype=jnp.int32)
    a = jnp.where(
        (ii[:, None] // W) == jnp.arange(NWC, dtype=jnp.int32)[None, :],
        jnp.exp2(jnp.float32(ii[:, None] % W)), 0.0).astype(jnp.bfloat16)
    grid = (B, S // S_TILE, N // NC)
    return pl.pallas_call(
        _pack_kernel,
        grid=grid,
        in_specs=[
            pl.BlockSpec((1, S_TILE, 128), lambda b, s, c: (b, s, 0)),
            pl.BlockSpec((1, 8, NC), lambda b, s, c: (b, 0, c)),
            pl.BlockSpec((NC, NWC), lambda b, s, c: (0, 0)),
        ],
        out_specs=pl.BlockSpec((1, S_TILE, NWC), lambda b, s, c: (b, s, c)),
        out_shape=jax.ShapeDtypeStruct((B, S, NWORDS), jnp.int32),
    )(q, p, a)


def _select_body(words_hbm, out_hbm, wv_ref, buf_ref, outv_ref):
    cid = lax.axis_index("c")
    sid = lax.axis_index("s")
    wid = sid * 2 + cid
    base = wid * ROWS_PER
    iota = lax.iota(jnp.int32, 16)

    def row_body(r, carry):
        row = base + r
        pltpu.sync_copy(words_hbm.at[row], wv_ref)

        def g_body(g, cnt):
            wvec = plsc.load_gather(wv_ref, [g * 16 + iota])
            nz = wvec != 0

            def w_cond(st):
                nzm, c = st
                return jnp.any(nzm) & (c < NSAMPLE)

            def w_step(st):
                nzm, c = st
                j = plsc.all_reduce_ffs(nzm)
                wsel = jnp.sum(jnp.where(iota == j, wvec, 0))
                bits = ((wsel >> iota) & 1) == 1
                incl = plsc.cumsum(jnp.where(bits, 1, 0))
                slot = c + incl - 1
                ok = bits & (slot < BUF)
                pidx = (g * 16 + j) * 16 + iota
                plsc.store_scatter(buf_ref, [slot], pidx, mask=ok)
                pc = jnp.sum(jnp.where(bits, 1, 0))
                return (nzm & (iota != j), c + pc)

            _, cnt = lax.while_loop(w_cond, w_step, (nz, cnt))
            return cnt

        count = lax.fori_loop(0, NWORDS // 16, g_body, 0)

        v0 = buf_ref[pl.ds(0, 16)]
        first_raw = jnp.sum(jnp.where(iota == 0, v0, 0))
        first = jnp.where(count > 0, first_raw, 0)
        for t in range(NSAMPLE // 16):
            vt = buf_ref[pl.ds(t * 16, 16)]
            slotv = t * 16 + iota
            outv_ref[pl.ds(t * 16, 16)] = jnp.where(slotv < count, vt, first)
        pltpu.sync_copy(outv_ref, out_hbm.at[row])
        return carry

    lax.fori_loop(0, ROWS_PER, row_body, 0)


def _select(words):
    mesh = plsc.VectorSubcoreMesh(core_axis_name="c", subcore_axis_name="s")
    cp = pltpu.CompilerParams()
    if "needs_layout_passes" in pltpu.CompilerParams.__dataclass_fields__:
        cp = dataclasses.replace(cp, needs_layout_passes=False)
    f = pl.kernel(
        _select_body,
        out_type=jax.ShapeDtypeStruct((ROWS, NSAMPLE), jnp.int32),
        mesh=mesh,
        scratch_types=[
            pltpu.VMEM((NWORDS,), jnp.int32),
            pltpu.VMEM((BUF,), jnp.int32),
            pltpu.VMEM((NSAMPLE,), jnp.int32),
        ],
        compiler_params=cp,
    )
    return f(words)


@jax.jit
def kernel(new_xyz, xyz):
    words = _pack(new_xyz, xyz).reshape(ROWS, NWORDS)
    inds = _select(words)
    return inds.reshape(B, S, NSAMPLE)


# speedup vs baseline: 70.9230x; 2.4686x over previous
import dataclasses
import functools

import jax
import jax.numpy as jnp
from jax import lax
from jax.experimental import pallas as pl
from jax.experimental.pallas import tpu as pltpu
from jax.experimental.pallas import tpu_sc as plsc

RADIUS2 = 0.1 * 0.1
NSAMPLE = 64
B, S, N = 4, 1024, 16384
W = 16
NWORDS = N // W
ROWS = B * S
S_TILE = 256
NC = 2048
NWC = NC // W

NUM_WORKERS = 32
ROWS_PER = ROWS // NUM_WORKERS
RB = 16
POSCAP = 96
BUF = 336


def _pack_kernel(q_ref, p_ref, a_ref, out_ref):
    q = q_ref[0]
    p = p_ref[0]
    qx, qy, qz = q[:, 0:1], q[:, 1:2], q[:, 2:3]
    px, py, pz = p[0:1, :], p[1:2, :], p[2:3, :]
    q2 = qx * qx + qy * qy + qz * qz
    p2 = px * px + py * py + pz * pz
    qb = q[:, 0:8].astype(jnp.bfloat16)
    pb = p.astype(jnp.bfloat16)
    qp = jnp.dot(qb, pb, preferred_element_type=jnp.float32)
    d2 = q2 + p2 - 2.0 * qp
    m = (d2 < RADIUS2).astype(jnp.bfloat16)
    w = jnp.dot(m, a_ref[...], preferred_element_type=jnp.float32)
    out_ref[0] = w.astype(jnp.int32)


def _pack(new_xyz, xyz):
    q = jnp.pad(new_xyz, ((0, 0), (0, 0), (0, 128 - 3)))
    p = jnp.pad(xyz.transpose(0, 2, 1), ((0, 0), (0, 8 - 3), (0, 0)))
    ii = jnp.arange(NC, dtype=jnp.int32)
    a = jnp.where(
        (ii[:, None] // W) == jnp.arange(NWC, dtype=jnp.int32)[None, :],
        jnp.exp2(jnp.float32(ii[:, None] % W)), 0.0).astype(jnp.bfloat16)
    grid = (B, S // S_TILE, N // NC)
    return pl.pallas_call(
        _pack_kernel,
        grid=grid,
        in_specs=[
            pl.BlockSpec((1, S_TILE, 128), lambda b, s, c: (b, s, 0)),
            pl.BlockSpec((1, 8, NC), lambda b, s, c: (b, 0, c)),
            pl.BlockSpec((NC, NWC), lambda b, s, c: (0, 0)),
        ],
        out_specs=pl.BlockSpec((1, S_TILE, NWC), lambda b, s, c: (b, s, c)),
        out_shape=jax.ShapeDtypeStruct((B, S, NWORDS), jnp.int32),
    )(q, p, a)


def _popcount16(w):
    x = w - ((w >> 1) & 0x5555)
    x = (x & 0x3333) + ((x >> 2) & 0x3333)
    x = (x + (x >> 4)) & 0x0F0F
    return (x + (x >> 8)) & 0x1F


def _select_body(words_hbm, out_hbm, wv_ref, pos_ref, buf_ref, outv_ref):
    cid = lax.axis_index("c")
    sid = lax.axis_index("s")
    wid = sid * 2 + cid
    base = wid * ROWS_PER
    iota = lax.iota(jnp.int32, 16)

    def blk_body(blk, carry0):
        row0 = base + blk * RB
        pltpu.sync_copy(words_hbm.at[pl.ds(row0, RB)], wv_ref)

        def row_body(r, carry):
            rowv = jnp.full((16,), r, jnp.int32)

            def g_body(g, pcnt):
                wvec = plsc.load_gather(wv_ref, [rowv, g * 16 + iota])
                nz = wvec != 0
                plsc.store_compressed(pos_ref.at[pl.ds(pcnt, 16)],
                                      g * 16 + iota, mask=nz)
                return pcnt + jnp.max(plsc.all_reduce_population_count(nz))

            pcnt = lax.fori_loop(0, NWORDS // 16, g_body, 0)
            npos = jnp.minimum(pcnt, POSCAP)

            def b_cond(st):
                t, cnt = st
                return (t * 16 < npos) & (cnt < NSAMPLE)

            def b_body(st):
                t, cnt = st
                lanes = t * 16 + iota
                posv = plsc.load_gather(pos_ref, [lanes])
                lanemask = lanes < npos
                posv = jnp.where(lanemask, posv, 0)
                wordv = plsc.load_gather(wv_ref, [rowv, posv])
                wordv = jnp.where(lanemask, wordv, 0)
                pc = _popcount16(wordv)
                incl = plsc.cumsum(pc)
                slot = cnt + incl - pc
                pbase = posv * 16
                for b in range(16):
                    a = (wordv >> b) & 1
                    plsc.store_scatter(buf_ref, [slot], pbase + b,
                                       mask=a == 1)
                    slot = slot + a
                return (t + 1, cnt + jnp.max(incl))

            _, count = lax.while_loop(b_cond, b_body, (0, 0))

            v0 = buf_ref[pl.ds(0, 16)]
            first_raw = jnp.sum(jnp.where(iota == 0, v0, 0))
            first = jnp.where(count > 0, first_raw, 0)
            for t in range(NSAMPLE // 16):
                vt = buf_ref[pl.ds(t * 16, 16)]
                slotv = t * 16 + iota
                plsc.store_scatter(outv_ref, [rowv, slotv],
                                   jnp.where(slotv < count, vt, first))
            return carry

        lax.fori_loop(0, RB, row_body, 0)
        pltpu.sync_copy(outv_ref, out_hbm.at[pl.ds(row0, RB)])
        return carry0

    lax.fori_loop(0, ROWS_PER // RB, blk_body, 0)


def _select(words):
    mesh = plsc.VectorSubcoreMesh(core_axis_name="c", subcore_axis_name="s")
    cp = pltpu.CompilerParams()
    if "needs_layout_passes" in pltpu.CompilerParams.__dataclass_fields__:
        cp = dataclasses.replace(cp, needs_layout_passes=False)
    f = pl.kernel(
        _select_body,
        out_type=jax.ShapeDtypeStruct((ROWS, NSAMPLE), jnp.int32),
        mesh=mesh,
        scratch_types=[
            pltpu.VMEM((RB, NWORDS), jnp.int32),
            pltpu.VMEM((NWORDS + 16,), jnp.int32),
            pltpu.VMEM((BUF,), jnp.int32),
            pltpu.VMEM((RB, NSAMPLE), jnp.int32),
        ],
        compiler_params=cp,
    )
    return f(words)


@jax.jit
def kernel(new_xyz, xyz):
    words = _pack(new_xyz, xyz).reshape(ROWS, NWORDS)
    inds = _select(words)
    return inds.reshape(B, S, NSAMPLE)


# speedup vs baseline: 98.3942x; 1.3873x over previous
import dataclasses
import functools

import jax
import jax.numpy as jnp
from jax import lax
from jax.experimental import pallas as pl
from jax.experimental.pallas import tpu as pltpu
from jax.experimental.pallas import tpu_sc as plsc

RADIUS2 = 0.1 * 0.1
NSAMPLE = 64
B, S, N = 4, 1024, 16384
W = 16
NWORDS = N // W
ROWS = B * S
S_TILE = 256
NC = 2048
NWC = NC // W

NUM_WORKERS = 32
ROWS_PER = ROWS // NUM_WORKERS
RB = 16
POSCAP = 96
BUF = 336


def _pack_kernel(q_ref, p_ref, a_ref, out_ref):
    q = q_ref[0]
    p = p_ref[0]
    qx, qy, qz = q[:, 0:1], q[:, 1:2], q[:, 2:3]
    px, py, pz = p[0:1, :], p[1:2, :], p[2:3, :]
    q2 = qx * qx + qy * qy + qz * qz
    p2 = px * px + py * py + pz * pz
    qb = q[:, 0:8].astype(jnp.bfloat16)
    pb = p.astype(jnp.bfloat16)
    qp = jnp.dot(qb, pb, preferred_element_type=jnp.float32)
    d2 = q2 + p2 - 2.0 * qp
    m = (d2 < RADIUS2).astype(jnp.bfloat16)
    w = jnp.dot(m, a_ref[...], preferred_element_type=jnp.float32)
    out_ref[0] = w.astype(jnp.int32)


def _pack_one(q, p, a):
    grid = (S // S_TILE, N // NC)
    out = pl.pallas_call(
        _pack_kernel,
        grid=grid,
        in_specs=[
            pl.BlockSpec((1, S_TILE, 128), lambda s, c: (0, s, 0)),
            pl.BlockSpec((1, 8, NC), lambda s, c: (0, 0, c)),
            pl.BlockSpec((NC, NWC), lambda s, c: (0, 0)),
        ],
        out_specs=pl.BlockSpec((1, S_TILE, NWC), lambda s, c: (0, s, c)),
        out_shape=jax.ShapeDtypeStruct((1, S, NWORDS), jnp.int32),
    )(q, p, a)
    return out.reshape(S, NWORDS)


def _popcount16(w):
    x = w - ((w >> 1) & 0x5555)
    x = (x & 0x3333) + ((x >> 2) & 0x3333)
    x = (x + (x >> 4)) & 0x0F0F
    return (x + (x >> 8)) & 0x1F


def _select_body(rows_per, words_hbm, out_hbm, wv_ref, pos_ref, buf_ref,
                 outv_ref):
    cid = lax.axis_index("c")
    sid = lax.axis_index("s")
    wid = sid * 2 + cid
    base = wid * rows_per
    iota = lax.iota(jnp.int32, 16)

    def blk_body(blk, carry0):
        row0 = base + blk * RB
        pltpu.sync_copy(words_hbm.at[pl.ds(row0, RB)], wv_ref)

        def row_body(r, carry):
            rowv = jnp.full((16,), r, jnp.int32)

            def g_body(g, pcnt):
                wvec = plsc.load_gather(wv_ref, [rowv, g * 16 + iota])
                nz = wvec != 0
                plsc.store_compressed(pos_ref.at[pl.ds(pcnt, 16)],
                                      g * 16 + iota, mask=nz)
                return pcnt + jnp.max(plsc.all_reduce_population_count(nz))

            pcnt = lax.fori_loop(0, NWORDS // 16, g_body, 0)
            npos = jnp.minimum(pcnt, POSCAP)

            def b_cond(st):
                t, cnt = st
                return (t * 16 < npos) & (cnt < NSAMPLE)

            def b_body(st):
                t, cnt = st
                lanes = t * 16 + iota
                posv = plsc.load_gather(pos_ref, [lanes])
                lanemask = lanes < npos
                posv = jnp.where(lanemask, posv, 0)
                wordv = plsc.load_gather(wv_ref, [rowv, posv])
                wordv = jnp.where(lanemask, wordv, 0)
                pc = _popcount16(wordv)
                incl = plsc.cumsum(pc)
                slot = cnt + incl - pc
                pbase = posv * 16
                for b in range(16):
                    a = (wordv >> b) & 1
                    plsc.store_scatter(buf_ref, [slot], pbase + b,
                                       mask=a == 1)
                    slot = slot + a
                return (t + 1, cnt + jnp.max(incl))

            _, count = lax.while_loop(b_cond, b_body, (0, 0))

            v0 = buf_ref[pl.ds(0, 16)]
            first_raw = jnp.sum(jnp.where(iota == 0, v0, 0))
            first = jnp.where(count > 0, first_raw, 0)
            for t in range(NSAMPLE // 16):
                vt = buf_ref[pl.ds(t * 16, 16)]
                slotv = t * 16 + iota
                plsc.store_scatter(outv_ref, [rowv, slotv],
                                   jnp.where(slotv < count, vt, first))
            return carry

        lax.fori_loop(0, RB, row_body, 0)
        pltpu.sync_copy(outv_ref, out_hbm.at[pl.ds(row0, RB)])
        return carry0

    lax.fori_loop(0, rows_per // RB, blk_body, 0)


def _select(words):
    rows = words.shape[0]
    mesh = plsc.VectorSubcoreMesh(core_axis_name="c", subcore_axis_name="s")
    cp = pltpu.CompilerParams()
    if "needs_layout_passes" in pltpu.CompilerParams.__dataclass_fields__:
        cp = dataclasses.replace(cp, needs_layout_passes=False)
    f = pl.kernel(
        functools.partial(_select_body, rows // NUM_WORKERS),
        out_type=jax.ShapeDtypeStruct((rows, NSAMPLE), jnp.int32),
        mesh=mesh,
        scratch_types=[
            pltpu.VMEM((RB, NWORDS), jnp.int32),
            pltpu.VMEM((NWORDS + 16,), jnp.int32),
            pltpu.VMEM((BUF,), jnp.int32),
            pltpu.VMEM((RB, NSAMPLE), jnp.int32),
        ],
        compiler_params=cp,
    )
    return f(words)


@jax.jit
def kernel(new_xyz, xyz):
    q = jnp.pad(new_xyz, ((0, 0), (0, 0), (0, 128 - 3)))
    p = jnp.pad(xyz.transpose(0, 2, 1), ((0, 0), (0, 8 - 3), (0, 0)))
    ii = jnp.arange(NC, dtype=jnp.int32)
    a = jnp.where(
        (ii[:, None] // W) == jnp.arange(NWC, dtype=jnp.int32)[None, :],
        jnp.exp2(jnp.float32(ii[:, None] % W)), 0.0).astype(jnp.bfloat16)
    outs = []
    for b in range(B):
        words = _pack_one(q[b:b + 1], p[b:b + 1], a)
        outs.append(_select(words))
    return jnp.stack(outs).reshape(B, S, NSAMPLE)


# speedup vs baseline: 98.5167x; 1.0012x over previous
import dataclasses
import functools

import jax
import jax.numpy as jnp
from jax import lax
from jax.experimental import pallas as pl
from jax.experimental.pallas import tpu as pltpu
from jax.experimental.pallas import tpu_sc as plsc

RADIUS2 = 0.1 * 0.1
NSAMPLE = 64
B, S, N = 4, 1024, 16384
W = 16
NWORDS = N // W
ROWS = B * S
S_TILE = 256
NC = 2048
NWC = NC // W

NUM_WORKERS = 32
RB = 16
POSCAP = 96
BUF = 64 + 16 * W


def _pack_kernel(q_ref, p_ref, a_ref, out_ref):
    q = q_ref[0]
    p = p_ref[0]
    qx, qy, qz = q[:, 0:1], q[:, 1:2], q[:, 2:3]
    px, py, pz = p[0:1, :], p[1:2, :], p[2:3, :]
    q2 = qx * qx + qy * qy + qz * qz
    p2 = px * px + py * py + pz * pz
    qb = q[:, 0:8].astype(jnp.bfloat16)
    pb = p.astype(jnp.bfloat16)
    qp = jnp.dot(qb, pb, preferred_element_type=jnp.float32)
    d2 = q2 + p2 - 2.0 * qp
    m = (d2 < RADIUS2).astype(jnp.bfloat16)
    w = jnp.dot(m, a_ref[...], preferred_element_type=jnp.float32)
    out_ref[0] = w.astype(jnp.int32)


def _pack_one(q, p, a):
    grid = (S // S_TILE, N // NC)
    out = pl.pallas_call(
        _pack_kernel,
        grid=grid,
        in_specs=[
            pl.BlockSpec((1, S_TILE, 128), lambda s, c: (0, s, 0)),
            pl.BlockSpec((1, 8, NC), lambda s, c: (0, 0, c)),
            pl.BlockSpec((NC, NWC), lambda s, c: (0, 0)),
        ],
        out_specs=pl.BlockSpec((1, S_TILE, NWC), lambda s, c: (0, s, c)),
        out_shape=jax.ShapeDtypeStruct((1, S, NWORDS), jnp.int32),
    )(q, p, a)
    return out.reshape(S, NWORDS)


def _popcount16(w):
    x = w - ((w >> 1) & 0x5555)
    x = (x & 0x3333) + ((x >> 2) & 0x3333)
    x = (x + (x >> 4)) & 0x0F0F
    return (x + (x >> 8)) & 0x1F


def _select_body(rows_per, words_hbm, out_hbm, wv_ref, pos_ref, buf_ref,
                 outv_ref):
    cid = lax.axis_index("c")
    sid = lax.axis_index("s")
    wid = sid * 2 + cid
    base = wid * rows_per
    iota = lax.iota(jnp.int32, 16)

    def blk_body(blk, carry0):
        row0 = base + blk * RB
        pltpu.sync_copy(words_hbm.at[pl.ds(row0, RB)], wv_ref)

        def row_body(r, carry):
            rowv = jnp.full((16,), r, jnp.int32)

            def g_body(g, pcnt):
                wvec = plsc.load_gather(wv_ref, [rowv, g * 16 + iota])
                nz = wvec != 0
                plsc.store_compressed(pos_ref.at[pl.ds(pcnt, 16)],
                                      g * 16 + iota, mask=nz)
                return pcnt + jnp.max(plsc.all_reduce_population_count(nz))

            pcnt = lax.fori_loop(0, NWORDS // 16, g_body, 0)
            npos = jnp.minimum(pcnt, POSCAP)

            def b_cond(st):
                t, cnt = st
                return (t * 16 < npos) & (cnt < NSAMPLE)

            def b_body(st):
                t, cnt = st
                lanes = t * 16 + iota
                posv = plsc.load_gather(pos_ref, [lanes])
                lanemask = lanes < npos
                posv = jnp.where(lanemask, posv, 0)
                wordv = plsc.load_gather(wv_ref, [rowv, posv])
                wordv = jnp.where(lanemask, wordv, 0)
                pc = _popcount16(wordv)
                incl = plsc.cumsum(pc)
                slot = cnt + incl - pc
                pbase = posv * W
                for b in range(W):
                    a = (wordv >> b) & 1
                    plsc.store_scatter(buf_ref, [slot], pbase + b,
                                       mask=a == 1)
                    slot = slot + a
                return (t + 1, cnt + jnp.max(incl))

            _, count = lax.while_loop(b_cond, b_body, (0, 0))

            v0 = buf_ref[pl.ds(0, 16)]
            first_raw = jnp.sum(jnp.where(iota == 0, v0, 0))
            first = jnp.where(count > 0, first_raw, 0)
            for t in range(NSAMPLE // 16):
                vt = buf_ref[pl.ds(t * 16, 16)]
                slotv = t * 16 + iota
                plsc.store_scatter(outv_ref, [rowv, slotv],
                                   jnp.where(slotv < count, vt, first))
            return carry

        lax.fori_loop(0, RB, row_body, 0)
        pltpu.sync_copy(outv_ref, out_hbm.at[pl.ds(row0, RB)])
        return carry0

    lax.fori_loop(0, rows_per // RB, blk_body, 0)


def _select(words):
    rows = words.shape[0]
    mesh = plsc.VectorSubcoreMesh(core_axis_name="c", subcore_axis_name="s")
    cp = pltpu.CompilerParams()
    if "needs_layout_passes" in pltpu.CompilerParams.__dataclass_fields__:
        cp = dataclasses.replace(cp, needs_layout_passes=False)
    f = pl.kernel(
        functools.partial(_select_body, rows // NUM_WORKERS),
        out_type=jax.ShapeDtypeStruct((rows, NSAMPLE), jnp.int32),
        mesh=mesh,
        scratch_types=[
            pltpu.VMEM((RB, NWORDS), jnp.int32),
            pltpu.VMEM((NWORDS + 16,), jnp.int32),
            pltpu.VMEM((BUF,), jnp.int32),
            pltpu.VMEM((RB, NSAMPLE), jnp.int32),
        ],
        compiler_params=cp,
    )
    return f(words)


def _prep(new_xyz, xyz):
    q = jnp.pad(new_xyz, ((0, 0), (0, 0), (0, 128 - 3)))
    p = jnp.pad(xyz.transpose(0, 2, 1), ((0, 0), (0, 8 - 3), (0, 0)))
    ii = jnp.arange(NC, dtype=jnp.int32)
    a = jnp.where(
        (ii[:, None] // W) == jnp.arange(NWC, dtype=jnp.int32)[None, :],
        jnp.exp2(jnp.float32(ii[:, None] % W)), 0.0).astype(jnp.bfloat16)
    return q, p, a


@jax.jit
def kernel(new_xyz, xyz):
    q, p, a = _prep(new_xyz, xyz)
    outs = []
    for b in range(B):
        words = _pack_one(q[b:b + 1], p[b:b + 1], a)
        outs.append(_select(words))
    return jnp.stack(outs).reshape(B, S, NSAMPLE)
